# baseline (speedup 1.0000x reference)
import jax
import jax.numpy as jnp
from jax.experimental import pallas as pl
from jax.experimental.pallas import tpu as pltpu

B, H, S, D = 8, 16, 4096, 128
Q = 16
BH = B * H


def _body(pos_ref, kval_ref, vval_ref, kc_ref, vc_ref, ko_ref, vo_ref):
    ko_ref[...] = kc_ref[...]
    vo_ref[...] = vc_ref[...]
    for q in range(Q):
        p = pos_ref[q]
        ko_ref[0, pl.ds(p, 1), :] = kval_ref[0, pl.ds(q, 1), :]
        vo_ref[0, pl.ds(p, 1), :] = vval_ref[0, pl.ds(q, 1), :]


def kernel(input_pos, k_val, v_val, k_cache, v_cache):
    kc = k_cache.reshape(BH, S, D)
    vc = v_cache.reshape(BH, S, D)
    kv = k_val.reshape(BH, Q, D)
    vv = v_val.reshape(BH, Q, D)
    grid = (BH,)
    cache_spec = pl.BlockSpec((1, S, D), lambda i: (i, 0, 0))
    val_spec = pl.BlockSpec((1, Q, D), lambda i: (i, 0, 0))
    ko, vo = pl.pallas_call(
        _body,
        grid=grid,
        in_specs=[
            pl.BlockSpec(memory_space=pltpu.SMEM),
            val_spec,
            val_spec,
            cache_spec,
            cache_spec,
        ],
        out_specs=[cache_spec, cache_spec],
        out_shape=[
            jax.ShapeDtypeStruct((BH, S, D), jnp.float32),
            jax.ShapeDtypeStruct((BH, S, D), jnp.float32),
        ],
        compiler_params=pltpu.CompilerParams(
            dimension_semantics=("arbitrary",),
        ),
    )(input_pos, kv, vv, kc, vc)
    return ko.reshape(B, H, S, D), vo.reshape(B, H, S, D)
